# Initial kernel scaffold; baseline (speedup 1.0000x reference)
import jax
import jax.numpy as jnp
from jax.experimental import pallas as pl


def kernel(x_flow, dst_ports, tcp_flags, tcp_flags_rev, ehf_src, ehf_dst, efh_src, efh_dst, eff_src, eff_dst, batch, emb_port, emb_tcp, emb_tcp_rev, W1_hf_rel, W1_hf_root, W1_fh_rel, W1_fh_root, W1_ff_rel, W1_ff_root, W2_hf_rel, W2_hf_root, W2_fh_rel, W2_fh_root, W2_ff_rel, W2_ff_root, b1_hf, b1_fh, b1_ff, b2_hf, b2_fh, b2_ff, Wc1, bc1, Wc2, bc2, Wc3, bc3):
    raise NotImplementedError("write your pallas kernel here")


# speedup vs baseline: 1.5539x; 1.5539x over previous
import functools

import jax
import jax.numpy as jnp
from jax import lax
from jax.experimental import pallas as pl
from jax.experimental.pallas import tpu as pltpu
from jax.experimental.pallas import tpu_sc as plsc

N_FLOW = 50000
N_HOST = 5000
E = 400000
FDIM = 108
HID = 128
NC = 10
B = 64

BLK = 2000
NBLK = N_FLOW // BLK
NF_PAD = (NBLK + 1) * BLK
NH_PAD = 5120

EB = 2048
NBATCH = 13
EPT = NBATCH * EB
E_PAD = 16 * EPT

PG_PER = 1664
PG_TOTAL = 32 * PG_PER

_mesh = lambda: plsc.VectorSubcoreMesh(core_axis_name="c", subcore_axis_name="s")
_SC_PARAMS = pltpu.CompilerParams(use_tc_tiling_on_sc=False)


def _make_port_gather():
  @functools.partial(
      pl.kernel,
      mesh=_mesh(),
      out_type=jax.ShapeDtypeStruct((32, PG_PER, 16), jnp.float32),
      compiler_params=_SC_PARAMS,
      scratch_types=[
          pltpu.VMEM((PG_PER,), jnp.int32),
          pltpu.VMEM((PG_PER, 16), jnp.float32),
          pltpu.SemaphoreType.DMA,
      ],
  )
  def gather_k(ports1, embp, out3, ibuf, gbuf, sem):
    cid = lax.axis_index("c")
    sid = lax.axis_index("s")
    wid = cid * 16 + sid
    pltpu.sync_copy(ports1.at[pl.ds(wid * PG_PER, PG_PER)], ibuf)
    pltpu.async_copy(embp.at[ibuf], gbuf, sem).wait()
    pltpu.sync_copy(gbuf, out3.at[wid])

  return gather_k


def _make_scatter(n_src8, n_dst):
  rpt = n_dst // 16

  @functools.partial(
      pl.kernel,
      mesh=_mesh(),
      out_type=jax.ShapeDtypeStruct((n_dst, 8, 16), jnp.float32),
      compiler_params=_SC_PARAMS,
      scratch_types=[
          pltpu.VMEM((EB,), jnp.int32),
          pltpu.VMEM((EB,), jnp.int32),
          pltpu.VMEM((EB,), jnp.int32),
          pltpu.VMEM((EB, 16), jnp.float32),
          pltpu.VMEM_SHARED((n_dst, 16), jnp.float32),
          pltpu.SemaphoreType.DMA,
      ],
  )
  def scatter_k(src1, dst1, x8, zhbm, out3, sbuf, dbuf, gidx, gbuf, acc, sem):
    cid = lax.axis_index("c")
    sid = lax.axis_index("s")

    edge_base = sid * EPT

    for k in range(4):
      c = cid * 4 + k

      pltpu.sync_copy(zhbm.at[pl.ds(0, rpt)], acc.at[pl.ds(sid * rpt, rpt)])
      plsc.subcore_barrier()

      def batch_body(b, carry):
        base = edge_base + b * EB
        pltpu.sync_copy(src1.at[pl.ds(base, EB)], sbuf)
        pltpu.sync_copy(dst1.at[pl.ds(base, EB)], dbuf)
        for j in range(EB // 16):
          v = sbuf[pl.ds(j * 16, 16)]
          gidx[pl.ds(j * 16, 16)] = v * 8 + c
        pltpu.async_copy(x8.at[gidx], gbuf, sem).wait()
        pltpu.sync_copy(gbuf, acc.at[dbuf], add=True)
        return carry

      lax.fori_loop(0, NBATCH, batch_body, 0)
      plsc.subcore_barrier()

      wb = sid * rpt
      pltpu.sync_copy(acc.at[pl.ds(wb, rpt)], out3.at[pl.ds(wb, rpt), c])
      plsc.subcore_barrier()

  return scatter_k


def _embed_tc(x_flow, port_rows, tcp3, tcp_rev3, et1, et2, A1):
  grid = NF_PAD // BLK

  def body(xf_ref, pr_ref, t1_ref, t2_ref, e1_ref, e2_ref, a1_ref,
           oxf_ref, op1_ref):
    i = pl.program_id(0)

    @pl.when(i < NBLK)
    def _():
      t1 = t1_ref[0]
      t2 = t2_ref[0]
      io = lax.broadcasted_iota(jnp.int32, (1, 256), 1)
      oh1 = (t1 == io).astype(jnp.float32)
      oh2 = (t2 == io).astype(jnp.float32)
      e1 = jnp.dot(oh1, e1_ref[...], preferred_element_type=jnp.float32)
      e2 = jnp.dot(oh2, e2_ref[...], preferred_element_type=jnp.float32)
      xf = jnp.concatenate([xf_ref[...], pr_ref[...], e1, e2], axis=1)
      oxf_ref[...] = xf
      op1_ref[...] = jnp.dot(xf, a1_ref[...], preferred_element_type=jnp.float32)

    @pl.when(i >= NBLK)
    def _():
      oxf_ref[...] = jnp.zeros((BLK, HID), jnp.float32)
      op1_ref[...] = jnp.zeros((BLK, HID), jnp.float32)

  clamp = lambda i: (jnp.minimum(i, NBLK - 1), 0)
  clamp3 = lambda i: (jnp.minimum(i, NBLK - 1), 0, 0)
  out = pl.pallas_call(
      body,
      grid=(grid,),
      in_specs=[
          pl.BlockSpec((BLK, FDIM), clamp),
          pl.BlockSpec((BLK, 16), clamp),
          pl.BlockSpec((1, BLK, 1), clamp3),
          pl.BlockSpec((1, BLK, 1), clamp3),
          pl.BlockSpec((256, 2), lambda i: (0, 0)),
          pl.BlockSpec((256, 2), lambda i: (0, 0)),
          pl.BlockSpec((HID, HID), lambda i: (0, 0)),
      ],
      out_specs=[
          pl.BlockSpec((BLK, HID), lambda i: (i, 0)),
          pl.BlockSpec((BLK, HID), lambda i: (i, 0)),
      ],
      out_shape=[
          jax.ShapeDtypeStruct((NF_PAD, HID), jnp.float32),
          jax.ShapeDtypeStruct((NF_PAD, HID), jnp.float32),
      ],
  )(x_flow, port_rows, tcp3, tcp_rev3, et1, et2, A1)
  return out


def _layer1_tc(P1, agg_ff1, W1_ff_rel, A2, c1):
  grid = NF_PAD // BLK

  def body(p1_ref, ag_ref, w_ref, a2_ref, c_ref, of1_ref, op2_ref):
    i = pl.program_id(0)

    @pl.when(i < NBLK)
    def _():
      f1 = p1_ref[...] + jnp.dot(ag_ref[...], w_ref[...],
                                 preferred_element_type=jnp.float32) + c_ref[...]
      f1 = jnp.maximum(f1, 0.0)
      of1_ref[...] = f1
      op2_ref[...] = jnp.dot(f1, a2_ref[...], preferred_element_type=jnp.float32)

    @pl.when(i >= NBLK)
    def _():
      of1_ref[...] = jnp.zeros((BLK, HID), jnp.float32)
      op2_ref[...] = jnp.zeros((BLK, HID), jnp.float32)

  clamp = lambda i: (jnp.minimum(i, NBLK - 1), 0)
  return pl.pallas_call(
      body,
      grid=(grid,),
      in_specs=[
          pl.BlockSpec((BLK, HID), lambda i: (i, 0)),
          pl.BlockSpec((BLK, HID), clamp),
          pl.BlockSpec((HID, HID), lambda i: (0, 0)),
          pl.BlockSpec((HID, HID), lambda i: (0, 0)),
          pl.BlockSpec((1, HID), lambda i: (0, 0)),
      ],
      out_specs=[
          pl.BlockSpec((BLK, HID), lambda i: (i, 0)),
          pl.BlockSpec((BLK, HID), lambda i: (i, 0)),
      ],
      out_shape=[
          jax.ShapeDtypeStruct((NF_PAD, HID), jnp.float32),
          jax.ShapeDtypeStruct((NF_PAD, HID), jnp.float32),
      ],
  )(P1, agg_ff1, W1_ff_rel, A2, c1)


def _host_tc(agg_fh, W1_fh_rel, b1_fh):
  HB = 320
  grid = NH_PAD // HB

  def body(ag_ref, w_ref, b_ref, out_ref):
    i = pl.program_id(0)
    h = jnp.dot(ag_ref[...], w_ref[...],
                preferred_element_type=jnp.float32) + b_ref[...]
    h = jnp.maximum(h, 0.0)
    row = i * HB + lax.broadcasted_iota(jnp.int32, (HB, 1), 0)
    out_ref[...] = jnp.where(row < N_HOST, h, 0.0)

  return pl.pallas_call(
      body,
      grid=(grid,),
      in_specs=[
          pl.BlockSpec((HB, HID), lambda i: (i, 0)),
          pl.BlockSpec((HID, HID), lambda i: (0, 0)),
          pl.BlockSpec((1, HID), lambda i: (0, 0)),
      ],
      out_specs=pl.BlockSpec((HB, HID), lambda i: (i, 0)),
      out_shape=jax.ShapeDtypeStruct((NH_PAD, HID), jnp.float32),
  )(agg_fh, W1_fh_rel, b1_fh)


def _layer2_pool_tc(P2, agg_ff2, agg_hf2, Wst, c2, batch3):
  def body(p2_ref, a1_ref, a2_ref, w_ref, c_ref, b_ref, out_ref):
    i = pl.program_id(0)

    @pl.when(i == 0)
    def _():
      out_ref[...] = jnp.full((B, HID), -jnp.inf, jnp.float32)

    cat = jnp.concatenate([a1_ref[...], a2_ref[...]], axis=1)
    f2 = p2_ref[...] + jnp.dot(cat, w_ref[...],
                               preferred_element_type=jnp.float32) + c_ref[...]
    bid = b_ref[0]
    lo = bid[0, 0]
    hi = bid[BLK - 1, 0]
    seg_iota = lax.broadcasted_iota(jnp.int32, (B, 1), 0)

    def seg_body(s, carry):
      m = jnp.max(jnp.where(bid == s, f2, -jnp.inf), axis=0, keepdims=True)
      upd = jnp.where(seg_iota == s, m, -jnp.inf)
      out_ref[...] = jnp.maximum(out_ref[...], upd)
      return carry

    lax.fori_loop(lo, hi + 1, seg_body, 0)

  return pl.pallas_call(
      body,
      grid=(NBLK,),
      in_specs=[
          pl.BlockSpec((BLK, HID), lambda i: (i, 0)),
          pl.BlockSpec((BLK, HID), lambda i: (i, 0)),
          pl.BlockSpec((BLK, HID), lambda i: (i, 0)),
          pl.BlockSpec((2 * HID, HID), lambda i: (0, 0)),
          pl.BlockSpec((1, HID), lambda i: (0, 0)),
          pl.BlockSpec((1, BLK, 1), lambda i: (i, 0, 0)),
      ],
      out_specs=pl.BlockSpec((B, HID), lambda i: (0, 0)),
      out_shape=jax.ShapeDtypeStruct((B, HID), jnp.float32),
  )(P2, agg_ff2, agg_hf2, Wst, c2, batch3)


def _mlp_tc(pooled, Wc1, bc1, Wc2, bc2, Wc3p, bc3p):
  def body(p_ref, w1_ref, b1_ref, w2_ref, b2_ref, w3_ref, b3_ref, out_ref):
    o = jnp.dot(p_ref[...], w1_ref[...], preferred_element_type=jnp.float32) + b1_ref[...]
    o = jnp.maximum(o, 0.0)
    o = jnp.dot(o, w2_ref[...], preferred_element_type=jnp.float32) + b2_ref[...]
    o = jnp.maximum(o, 0.0)
    out_ref[...] = jnp.dot(o, w3_ref[...], preferred_element_type=jnp.float32) + b3_ref[...]

  return pl.pallas_call(
      body,
      out_shape=jax.ShapeDtypeStruct((B, HID), jnp.float32),
  )(pooled, Wc1, bc1, Wc2, bc2, Wc3p, bc3p)


def _pad_edges(src, dst, pad_src):
  npad = E_PAD - E
  src_p = jnp.concatenate(
      [src.astype(jnp.int32), jnp.full((npad,), pad_src, jnp.int32)])
  dst_p = jnp.concatenate(
      [dst.astype(jnp.int32), jnp.zeros((npad,), jnp.int32)])
  return src_p, dst_p


def kernel(x_flow, dst_ports, tcp_flags, tcp_flags_rev, ehf_src, ehf_dst,
           efh_src, efh_dst, eff_src, eff_dst, batch, emb_port, emb_tcp,
           emb_tcp_rev, W1_hf_rel, W1_hf_root, W1_fh_rel, W1_fh_root,
           W1_ff_rel, W1_ff_root, W2_hf_rel, W2_hf_root, W2_fh_rel,
           W2_fh_root, W2_ff_rel, W2_ff_root, b1_hf, b1_fh, b1_ff, b2_hf,
           b2_fh, b2_ff, Wc1, bc1, Wc2, bc2, Wc3, bc3):
  f32 = jnp.float32

  A1 = (W1_hf_root + W1_ff_root).astype(f32)
  c1 = (b1_hf + b1_ff).reshape(1, HID).astype(f32)
  A2 = (W2_hf_root + W2_ff_root).astype(f32)
  c2 = (b2_hf + b2_ff).reshape(1, HID).astype(f32)
  Wst = jnp.concatenate([W2_ff_rel, W2_hf_rel], axis=0).astype(f32)
  Wc3p = jnp.zeros((HID, HID), f32).at[:, :NC].set(Wc3)
  bc3p = jnp.zeros((1, HID), f32).at[0, :NC].set(bc3)

  ports_p = jnp.concatenate([
      dst_ports.astype(jnp.int32),
      jnp.zeros((PG_TOTAL - N_FLOW,), jnp.int32)])
  tcp3 = tcp_flags.astype(jnp.int32).reshape(NBLK, BLK, 1)
  tcp_rev3 = tcp_flags_rev.astype(jnp.int32).reshape(NBLK, BLK, 1)
  batch3 = batch.astype(jnp.int32).reshape(NBLK, BLK, 1)

  eff_s3, eff_d3 = _pad_edges(eff_src, eff_dst, N_FLOW)
  efh_s3, efh_d3 = _pad_edges(efh_src, efh_dst, N_FLOW)
  ehf_s3, ehf_d3 = _pad_edges(ehf_src, ehf_dst, N_HOST)

  port4 = _make_port_gather()(ports_p, emb_port)
  port_rows = port4.reshape(PG_TOTAL, 16)

  xf_pad, P1 = _embed_tc(x_flow, port_rows, tcp3, tcp_rev3,
                         emb_tcp, emb_tcp_rev, A1)
  xf8 = xf_pad.reshape(NF_PAD * 8, 16)

  zhbm = jnp.zeros((N_FLOW // 16, 16), f32)

  scat_flow = _make_scatter(NF_PAD * 8, N_FLOW)
  scat_host = _make_scatter(NF_PAD * 8, NH_PAD)
  agg_ff1 = scat_flow(eff_s3, eff_d3, xf8, zhbm).reshape(N_FLOW, HID)
  agg_fh = scat_host(efh_s3, efh_d3, xf8, zhbm).reshape(NH_PAD, HID)

  f1_pad, P2 = _layer1_tc(P1, agg_ff1, W1_ff_rel, A2, c1)
  h1_pad = _host_tc(agg_fh, W1_fh_rel, b1_fh.reshape(1, HID))

  f18 = f1_pad.reshape(NF_PAD * 8, 16)
  h18 = h1_pad.reshape(NH_PAD * 8, 16)
  agg_ff2 = scat_flow(eff_s3, eff_d3, f18, zhbm).reshape(N_FLOW, HID)
  scat_hf = _make_scatter(NH_PAD * 8, N_FLOW)
  agg_hf2 = scat_hf(ehf_s3, ehf_d3, h18, zhbm).reshape(N_FLOW, HID)

  pooled = _layer2_pool_tc(P2, agg_ff2, agg_hf2, Wst, c2, batch3)
  out = _mlp_tc(pooled, Wc1, bc1.reshape(1, HID), Wc2, bc2.reshape(1, HID),
                Wc3p, bc3p)
  return out[:, :NC]


# speedup vs baseline: 2.6865x; 1.7289x over previous
import functools

import jax
import jax.numpy as jnp
from jax import lax
from jax.experimental import pallas as pl
from jax.experimental.pallas import tpu as pltpu
from jax.experimental.pallas import tpu_sc as plsc

N_FLOW = 50000
N_HOST = 5000
E = 400000
FDIM = 108
HID = 128
NC = 10
B = 64

BLK = 2000
NBLK = N_FLOW // BLK
NF_PAD = (NBLK + 1) * BLK
NH_PAD = 6000

EB = 1024
NBATCH = 26
EPT = NBATCH * EB
E_PAD = 16 * EPT

PG_PER = 1664
PG_TOTAL = 32 * PG_PER

_mesh = lambda: plsc.VectorSubcoreMesh(core_axis_name="c", subcore_axis_name="s")
_SC_PARAMS = pltpu.CompilerParams(use_tc_tiling_on_sc=False)


def _make_port_gather():
  @functools.partial(
      pl.kernel,
      mesh=_mesh(),
      out_type=jax.ShapeDtypeStruct((32, PG_PER, 16), jnp.float32),
      compiler_params=_SC_PARAMS,
      scratch_types=[
          pltpu.VMEM((PG_PER,), jnp.int32),
          pltpu.VMEM((PG_PER, 16), jnp.float32),
          pltpu.SemaphoreType.DMA,
      ],
  )
  def gather_k(ports1, embp, out3, ibuf, gbuf, sem):
    cid = lax.axis_index("c")
    sid = lax.axis_index("s")
    wid = cid * 16 + sid
    pltpu.sync_copy(ports1.at[pl.ds(wid * PG_PER, PG_PER)], ibuf)
    pltpu.async_copy(embp.at[ibuf], gbuf, sem).wait()
    pltpu.sync_copy(gbuf, out3.at[wid])

  return gather_k


def _make_scatter(n_src4, n_dst):
  rpt = n_dst // 16

  @functools.partial(
      pl.kernel,
      mesh=_mesh(),
      out_type=jax.ShapeDtypeStruct((n_dst, 4, 32), jnp.bfloat16),
      compiler_params=_SC_PARAMS,
      scratch_types=[
          pltpu.VMEM((EB,), jnp.int32),
          pltpu.VMEM((EB,), jnp.int32),
          pltpu.VMEM((EB,), jnp.int32),
          pltpu.VMEM((EB,), jnp.int32),
          pltpu.VMEM((EB, 32), jnp.bfloat16),
          pltpu.VMEM((EB, 32), jnp.bfloat16),
          pltpu.VMEM_SHARED((n_dst, 32), jnp.bfloat16),
          pltpu.SemaphoreType.DMA,
          pltpu.SemaphoreType.DMA,
          pltpu.SemaphoreType.DMA,
          pltpu.SemaphoreType.DMA,
      ],
  )
  def scatter_k(src4, dst1, x4, zhbm, out3, gi0, gi1, db0, db1, gb0, gb1,
                acc, sg0, sg1, ss0, ss1):
    cid = lax.axis_index("c")
    sid = lax.axis_index("s")
    edge_base = sid * EPT

    gidxs, dbufs = (gi0, gi1), (db0, db1)
    gbufs, gsems, ssems = (gb0, gb1), (sg0, sg1), (ss0, ss1)

    for k in range(2):
      c = cid * 2 + k

      pltpu.sync_copy(zhbm.at[pl.ds(0, rpt)], acc.at[pl.ds(sid * rpt, rpt)])
      plsc.subcore_barrier()

      def load_and_start_gather(b, slot):
        base = edge_base + b * EB
        pltpu.sync_copy(src4.at[pl.ds(base, EB)], gidxs[slot])
        pltpu.sync_copy(dst1.at[pl.ds(base, EB)], dbufs[slot])
        for j in range(EB // 16):
          v = gidxs[slot][pl.ds(j * 16, 16)]
          gidxs[slot][pl.ds(j * 16, 16)] = v + c
        pltpu.async_copy(x4.at[gidxs[slot]], gbufs[slot], gsems[slot])

      def wait_gather(slot):
        pltpu.make_async_copy(x4.at[gidxs[slot]], gbufs[slot],
                              gsems[slot]).wait()

      def scatter_desc(slot):
        return pltpu.make_async_copy(gbufs[slot], acc.at[dbufs[slot]],
                                     ssems[slot])

      load_and_start_gather(0, 0)

      def pair_body(i, carry):
        for slot in range(2):
          other = 1 - slot
          b = 2 * i + slot
          wait_gather(slot)
          scatter_desc(slot).start(add=True)

          @pl.when(b + 1 < NBATCH)
          def _():
            @pl.when(b >= 1)
            def _():
              scatter_desc(other).wait()
            load_and_start_gather(b + 1, other)
        return carry

      lax.fori_loop(0, NBATCH // 2, pair_body, 0)
      scatter_desc(0).wait()
      scatter_desc(1).wait()
      plsc.subcore_barrier()

      wb = sid * rpt
      pltpu.sync_copy(acc.at[pl.ds(wb, rpt)], out3.at[pl.ds(wb, rpt), c])
      plsc.subcore_barrier()

  return scatter_k


def _embed_tc(x_flow, port_rows, tcp3, tcp_rev3, et1, et2, A1):
  grid = NF_PAD // BLK

  def body(xf_ref, pr_ref, t1_ref, t2_ref, e1_ref, e2_ref, a1_ref,
           oxf_ref, op1_ref):
    i = pl.program_id(0)

    @pl.when(i < NBLK)
    def _():
      t1 = t1_ref[0]
      t2 = t2_ref[0]
      io = lax.broadcasted_iota(jnp.int32, (1, 256), 1)
      oh1 = (t1 == io).astype(jnp.float32)
      oh2 = (t2 == io).astype(jnp.float32)
      e1 = jnp.dot(oh1, e1_ref[...], preferred_element_type=jnp.float32)
      e2 = jnp.dot(oh2, e2_ref[...], preferred_element_type=jnp.float32)
      xf = jnp.concatenate([xf_ref[...], pr_ref[...], e1, e2], axis=1)
      oxf_ref[...] = xf.astype(jnp.bfloat16)
      op1_ref[...] = jnp.dot(xf, a1_ref[...], preferred_element_type=jnp.float32)

    @pl.when(i >= NBLK)
    def _():
      oxf_ref[...] = jnp.zeros((BLK, HID), jnp.bfloat16)
      op1_ref[...] = jnp.zeros((BLK, HID), jnp.float32)

  clamp = lambda i: (jnp.minimum(i, NBLK - 1), 0)
  clamp3 = lambda i: (jnp.minimum(i, NBLK - 1), 0, 0)
  out = pl.pallas_call(
      body,
      grid=(grid,),
      in_specs=[
          pl.BlockSpec((BLK, FDIM), clamp),
          pl.BlockSpec((BLK, 16), clamp),
          pl.BlockSpec((1, BLK, 1), clamp3),
          pl.BlockSpec((1, BLK, 1), clamp3),
          pl.BlockSpec((256, 2), lambda i: (0, 0)),
          pl.BlockSpec((256, 2), lambda i: (0, 0)),
          pl.BlockSpec((HID, HID), lambda i: (0, 0)),
      ],
      out_specs=[
          pl.BlockSpec((BLK, HID), lambda i: (i, 0)),
          pl.BlockSpec((BLK, HID), lambda i: (i, 0)),
      ],
      out_shape=[
          jax.ShapeDtypeStruct((NF_PAD, HID), jnp.bfloat16),
          jax.ShapeDtypeStruct((NF_PAD, HID), jnp.float32),
      ],
  )(x_flow, port_rows, tcp3, tcp_rev3, et1, et2, A1)
  return out


def _layer1_tc(P1, agg_ff1, W1_ff_rel, A2, c1):
  grid = NF_PAD // BLK

  def body(p1_ref, ag_ref, w_ref, a2_ref, c_ref, of1_ref, op2_ref):
    i = pl.program_id(0)

    @pl.when(i < NBLK)
    def _():
      ag = ag_ref[...].astype(jnp.float32)
      f1 = p1_ref[...] + jnp.dot(ag, w_ref[...],
                                 preferred_element_type=jnp.float32) + c_ref[...]
      f1 = jnp.maximum(f1, 0.0)
      of1_ref[...] = f1.astype(jnp.bfloat16)
      op2_ref[...] = jnp.dot(f1, a2_ref[...], preferred_element_type=jnp.float32)

    @pl.when(i >= NBLK)
    def _():
      of1_ref[...] = jnp.zeros((BLK, HID), jnp.bfloat16)
      op2_ref[...] = jnp.zeros((BLK, HID), jnp.float32)

  clamp = lambda i: (jnp.minimum(i, NBLK - 1), 0)
  return pl.pallas_call(
      body,
      grid=(grid,),
      in_specs=[
          pl.BlockSpec((BLK, HID), lambda i: (i, 0)),
          pl.BlockSpec((BLK, HID), clamp),
          pl.BlockSpec((HID, HID), lambda i: (0, 0)),
          pl.BlockSpec((HID, HID), lambda i: (0, 0)),
          pl.BlockSpec((1, HID), lambda i: (0, 0)),
      ],
      out_specs=[
          pl.BlockSpec((BLK, HID), lambda i: (i, 0)),
          pl.BlockSpec((BLK, HID), lambda i: (i, 0)),
      ],
      out_shape=[
          jax.ShapeDtypeStruct((NF_PAD, HID), jnp.bfloat16),
          jax.ShapeDtypeStruct((NF_PAD, HID), jnp.float32),
      ],
  )(P1, agg_ff1, W1_ff_rel, A2, c1)


def _host_tc(agg_fh, W1_fh_rel, b1_fh):
  HB = BLK
  grid = NH_PAD // HB

  def body(ag_ref, w_ref, b_ref, out_ref):
    i = pl.program_id(0)
    ag = ag_ref[...].astype(jnp.float32)
    h = jnp.dot(ag, w_ref[...],
                preferred_element_type=jnp.float32) + b_ref[...]
    h = jnp.maximum(h, 0.0)
    row = i * HB + lax.broadcasted_iota(jnp.int32, (HB, 1), 0)
    out_ref[...] = jnp.where(row < N_HOST, h, 0.0).astype(jnp.bfloat16)

  return pl.pallas_call(
      body,
      grid=(grid,),
      in_specs=[
          pl.BlockSpec((HB, HID), lambda i: (i, 0)),
          pl.BlockSpec((HID, HID), lambda i: (0, 0)),
          pl.BlockSpec((1, HID), lambda i: (0, 0)),
      ],
      out_specs=pl.BlockSpec((HB, HID), lambda i: (i, 0)),
      out_shape=jax.ShapeDtypeStruct((NH_PAD, HID), jnp.bfloat16),
  )(agg_fh, W1_fh_rel, b1_fh)


def _layer2_pool_tc(P2, agg_ff2, agg_hf2, Wst, c2, batch3):
  def body(p2_ref, a1_ref, a2_ref, w_ref, c_ref, b_ref, out_ref):
    i = pl.program_id(0)

    @pl.when(i == 0)
    def _():
      out_ref[...] = jnp.full((B, HID), -jnp.inf, jnp.float32)

    cat = jnp.concatenate([a1_ref[...].astype(jnp.float32),
                           a2_ref[...].astype(jnp.float32)], axis=1)
    f2 = p2_ref[...] + jnp.dot(cat, w_ref[...],
                               preferred_element_type=jnp.float32) + c_ref[...]
    bid = b_ref[0]
    lo = bid[0, 0]
    hi = bid[BLK - 1, 0]
    seg_iota = lax.broadcasted_iota(jnp.int32, (B, 1), 0)

    def seg_body(s, carry):
      m = jnp.max(jnp.where(bid == s, f2, -jnp.inf), axis=0, keepdims=True)
      upd = jnp.where(seg_iota == s, m, -jnp.inf)
      out_ref[...] = jnp.maximum(out_ref[...], upd)
      return carry

    lax.fori_loop(lo, hi + 1, seg_body, 0)

  return pl.pallas_call(
      body,
      grid=(NBLK,),
      in_specs=[
          pl.BlockSpec((BLK, HID), lambda i: (i, 0)),
          pl.BlockSpec((BLK, HID), lambda i: (i, 0)),
          pl.BlockSpec((BLK, HID), lambda i: (i, 0)),
          pl.BlockSpec((2 * HID, HID), lambda i: (0, 0)),
          pl.BlockSpec((1, HID), lambda i: (0, 0)),
          pl.BlockSpec((1, BLK, 1), lambda i: (i, 0, 0)),
      ],
      out_specs=pl.BlockSpec((B, HID), lambda i: (0, 0)),
      out_shape=jax.ShapeDtypeStruct((B, HID), jnp.float32),
  )(P2, agg_ff2, agg_hf2, Wst, c2, batch3)


def _mlp_tc(pooled, Wc1, bc1, Wc2, bc2, Wc3p, bc3p):
  def body(p_ref, w1_ref, b1_ref, w2_ref, b2_ref, w3_ref, b3_ref, out_ref):
    o = jnp.dot(p_ref[...], w1_ref[...], preferred_element_type=jnp.float32) + b1_ref[...]
    o = jnp.maximum(o, 0.0)
    o = jnp.dot(o, w2_ref[...], preferred_element_type=jnp.float32) + b2_ref[...]
    o = jnp.maximum(o, 0.0)
    out_ref[...] = jnp.dot(o, w3_ref[...], preferred_element_type=jnp.float32) + b3_ref[...]

  return pl.pallas_call(
      body,
      out_shape=jax.ShapeDtypeStruct((B, HID), jnp.float32),
  )(pooled, Wc1, bc1, Wc2, bc2, Wc3p, bc3p)


def _pad_edges(src, dst, pad_src):
  npad = E_PAD - E
  src_p = jnp.concatenate(
      [src.astype(jnp.int32), jnp.full((npad,), pad_src, jnp.int32)])
  dst_p = jnp.concatenate(
      [dst.astype(jnp.int32), jnp.zeros((npad,), jnp.int32)])
  return src_p * 4, dst_p


def kernel(x_flow, dst_ports, tcp_flags, tcp_flags_rev, ehf_src, ehf_dst,
           efh_src, efh_dst, eff_src, eff_dst, batch, emb_port, emb_tcp,
           emb_tcp_rev, W1_hf_rel, W1_hf_root, W1_fh_rel, W1_fh_root,
           W1_ff_rel, W1_ff_root, W2_hf_rel, W2_hf_root, W2_fh_rel,
           W2_fh_root, W2_ff_rel, W2_ff_root, b1_hf, b1_fh, b1_ff, b2_hf,
           b2_fh, b2_ff, Wc1, bc1, Wc2, bc2, Wc3, bc3):
  f32 = jnp.float32

  A1 = (W1_hf_root + W1_ff_root).astype(f32)
  c1 = (b1_hf + b1_ff).reshape(1, HID).astype(f32)
  A2 = (W2_hf_root + W2_ff_root).astype(f32)
  c2 = (b2_hf + b2_ff).reshape(1, HID).astype(f32)
  Wst = jnp.concatenate([W2_ff_rel, W2_hf_rel], axis=0).astype(f32)
  Wc3p = jnp.zeros((HID, HID), f32).at[:, :NC].set(Wc3)
  bc3p = jnp.zeros((1, HID), f32).at[0, :NC].set(bc3)

  ports_p = jnp.concatenate([
      dst_ports.astype(jnp.int32),
      jnp.zeros((PG_TOTAL - N_FLOW,), jnp.int32)])
  tcp3 = tcp_flags.astype(jnp.int32).reshape(NBLK, BLK, 1)
  tcp_rev3 = tcp_flags_rev.astype(jnp.int32).reshape(NBLK, BLK, 1)
  batch3 = batch.astype(jnp.int32).reshape(NBLK, BLK, 1)

  eff_s3, eff_d3 = _pad_edges(eff_src, eff_dst, N_FLOW)
  efh_s3, efh_d3 = _pad_edges(efh_src, efh_dst, N_FLOW)
  ehf_s3, ehf_d3 = _pad_edges(ehf_src, ehf_dst, N_HOST)

  port4 = _make_port_gather()(ports_p, emb_port)
  port_rows = port4.reshape(PG_TOTAL, 16)

  xf_bf, P1 = _embed_tc(x_flow, port_rows, tcp3, tcp_rev3,
                        emb_tcp, emb_tcp_rev, A1)
  xf2 = xf_bf.reshape(NF_PAD * 4, 32)

  zhbm = jnp.zeros((N_FLOW // 16, 32), jnp.bfloat16)

  scat_flow = _make_scatter(NF_PAD * 4, N_FLOW)
  scat_host = _make_scatter(NF_PAD * 4, NH_PAD)
  agg_ff1 = scat_flow(eff_s3, eff_d3, xf2, zhbm).reshape(N_FLOW, HID)
  agg_fh = scat_host(efh_s3, efh_d3, xf2, zhbm).reshape(NH_PAD, HID)

  f1_bf, P2 = _layer1_tc(P1, agg_ff1, W1_ff_rel, A2, c1)
  h1_bf = _host_tc(agg_fh, W1_fh_rel, b1_fh.reshape(1, HID))

  f12 = f1_bf.reshape(NF_PAD * 4, 32)
  h12 = h1_bf.reshape(NH_PAD * 4, 32)
  agg_ff2 = scat_flow(eff_s3, eff_d3, f12, zhbm).reshape(N_FLOW, HID)
  scat_hf = _make_scatter(NH_PAD * 4, N_FLOW)
  agg_hf2 = scat_hf(ehf_s3, ehf_d3, h12, zhbm).reshape(N_FLOW, HID)

  pooled = _layer2_pool_tc(P2, agg_ff2, agg_hf2, Wst, c2, batch3)
  out = _mlp_tc(pooled, Wc1, bc1.reshape(1, HID), Wc2, bc2.reshape(1, HID),
                Wc3p, bc3p)
  return out[:, :NC]


# speedup vs baseline: 2.7152x; 1.0107x over previous
import functools

import jax
import jax.numpy as jnp
from jax import lax
from jax.experimental import pallas as pl
from jax.experimental.pallas import tpu as pltpu
from jax.experimental.pallas import tpu_sc as plsc

N_FLOW = 50000
N_HOST = 5000
E = 400000
FDIM = 108
HID = 128
NC = 10
B = 64

BLK = 2000
NBLK = N_FLOW // BLK
NF_PAD = (NBLK + 1) * BLK
NH_PAD = 6000

EB = 1024
NBATCH = 26
EPT = NBATCH * EB
E_PAD = 16 * EPT

PG_PER = 1664
PG_TOTAL = 32 * PG_PER

_mesh = lambda: plsc.VectorSubcoreMesh(core_axis_name="c", subcore_axis_name="s")
_SC_PARAMS = pltpu.CompilerParams(use_tc_tiling_on_sc=False)


def _make_port_gather():
  @functools.partial(
      pl.kernel,
      mesh=_mesh(),
      out_type=jax.ShapeDtypeStruct((32, PG_PER, 16), jnp.float32),
      compiler_params=_SC_PARAMS,
      scratch_types=[
          pltpu.VMEM((PG_PER,), jnp.int32),
          pltpu.VMEM((PG_PER, 16), jnp.float32),
          pltpu.SemaphoreType.DMA,
      ],
  )
  def gather_k(ports1, embp, out3, ibuf, gbuf, sem):
    cid = lax.axis_index("c")
    sid = lax.axis_index("s")
    wid = cid * 16 + sid
    pltpu.sync_copy(ports1.at[pl.ds(wid * PG_PER, PG_PER)], ibuf)
    pltpu.async_copy(embp.at[ibuf], gbuf, sem).wait()
    pltpu.sync_copy(gbuf, out3.at[wid])

  return gather_k


def _make_scatter(n_dst, cw):
  rpt = n_dst // 16
  nch = 128 // cw
  kpc = nch // 2

  @functools.partial(
      pl.kernel,
      mesh=_mesh(),
      out_type=jax.ShapeDtypeStruct((n_dst, nch, cw), jnp.bfloat16),
      compiler_params=_SC_PARAMS,
      scratch_types=[
          pltpu.VMEM((EB,), jnp.int32),
          pltpu.VMEM((EB,), jnp.int32),
          pltpu.VMEM((EB,), jnp.int32),
          pltpu.VMEM((EB,), jnp.int32),
          pltpu.VMEM((EB, cw), jnp.bfloat16),
          pltpu.VMEM((EB, cw), jnp.bfloat16),
          pltpu.VMEM_SHARED((n_dst, cw), jnp.bfloat16),
          pltpu.SemaphoreType.DMA,
          pltpu.SemaphoreType.DMA,
          pltpu.SemaphoreType.DMA,
          pltpu.SemaphoreType.DMA,
      ],
  )
  def scatter_k(srcv, dst1, xv, zhbm, out3, gi0, gi1, db0, db1, gb0, gb1,
                acc, sg0, sg1, ss0, ss1):
    cid = lax.axis_index("c")
    sid = lax.axis_index("s")
    edge_base = sid * EPT

    gidxs, dbufs = (gi0, gi1), (db0, db1)
    gbufs, gsems, ssems = (gb0, gb1), (sg0, sg1), (ss0, ss1)

    for k in range(kpc):
      c = cid * kpc + k

      pltpu.sync_copy(zhbm.at[pl.ds(0, rpt)], acc.at[pl.ds(sid * rpt, rpt)])
      plsc.subcore_barrier()

      def load_and_start_gather(b, slot):
        base = edge_base + b * EB
        pltpu.sync_copy(srcv.at[pl.ds(base, EB)], gidxs[slot])
        pltpu.sync_copy(dst1.at[pl.ds(base, EB)], dbufs[slot])
        for j in range(EB // 16):
          v = gidxs[slot][pl.ds(j * 16, 16)]
          gidxs[slot][pl.ds(j * 16, 16)] = v + c
        pltpu.async_copy(xv.at[gidxs[slot]], gbufs[slot], gsems[slot])

      def wait_gather(slot):
        pltpu.make_async_copy(xv.at[gidxs[slot]], gbufs[slot],
                              gsems[slot]).wait()

      def scatter_desc(slot):
        return pltpu.make_async_copy(gbufs[slot], acc.at[dbufs[slot]],
                                     ssems[slot])

      load_and_start_gather(0, 0)

      def pair_body(i, carry):
        for slot in range(2):
          other = 1 - slot
          b = 2 * i + slot
          wait_gather(slot)
          scatter_desc(slot).start(add=True)

          @pl.when(b + 1 < NBATCH)
          def _():
            @pl.when(b >= 1)
            def _():
              scatter_desc(other).wait()
            load_and_start_gather(b + 1, other)
        return carry

      lax.fori_loop(0, NBATCH // 2, pair_body, 0)
      scatter_desc(0).wait()
      scatter_desc(1).wait()
      plsc.subcore_barrier()

      wb = sid * rpt
      pltpu.sync_copy(acc.at[pl.ds(wb, rpt)], out3.at[pl.ds(wb, rpt), c])
      plsc.subcore_barrier()

  return scatter_k


def _embed_tc(x_flow, port_rows, tcp3, tcp_rev3, et1, et2, A1):
  grid = NF_PAD // BLK

  def body(xf_ref, pr_ref, t1_ref, t2_ref, e1_ref, e2_ref, a1_ref,
           oxf_ref, op1_ref):
    i = pl.program_id(0)

    @pl.when(i < NBLK)
    def _():
      t1 = t1_ref[0]
      t2 = t2_ref[0]
      io = lax.broadcasted_iota(jnp.int32, (1, 256), 1)
      oh1 = (t1 == io).astype(jnp.float32)
      oh2 = (t2 == io).astype(jnp.float32)
      e1 = jnp.dot(oh1, e1_ref[...], preferred_element_type=jnp.float32)
      e2 = jnp.dot(oh2, e2_ref[...], preferred_element_type=jnp.float32)
      xf = jnp.concatenate([xf_ref[...], pr_ref[...], e1, e2], axis=1)
      oxf_ref[...] = xf.astype(jnp.bfloat16)
      op1_ref[...] = jnp.dot(xf, a1_ref[...], preferred_element_type=jnp.float32)

    @pl.when(i >= NBLK)
    def _():
      oxf_ref[...] = jnp.zeros((BLK, HID), jnp.bfloat16)
      op1_ref[...] = jnp.zeros((BLK, HID), jnp.float32)

  clamp = lambda i: (jnp.minimum(i, NBLK - 1), 0)
  clamp3 = lambda i: (jnp.minimum(i, NBLK - 1), 0, 0)
  out = pl.pallas_call(
      body,
      grid=(grid,),
      in_specs=[
          pl.BlockSpec((BLK, FDIM), clamp),
          pl.BlockSpec((BLK, 16), clamp),
          pl.BlockSpec((1, BLK, 1), clamp3),
          pl.BlockSpec((1, BLK, 1), clamp3),
          pl.BlockSpec((256, 2), lambda i: (0, 0)),
          pl.BlockSpec((256, 2), lambda i: (0, 0)),
          pl.BlockSpec((HID, HID), lambda i: (0, 0)),
      ],
      out_specs=[
          pl.BlockSpec((BLK, HID), lambda i: (i, 0)),
          pl.BlockSpec((BLK, HID), lambda i: (i, 0)),
      ],
      out_shape=[
          jax.ShapeDtypeStruct((NF_PAD, HID), jnp.bfloat16),
          jax.ShapeDtypeStruct((NF_PAD, HID), jnp.float32),
      ],
  )(x_flow, port_rows, tcp3, tcp_rev3, et1, et2, A1)
  return out


def _layer1_tc(P1, agg_ff1, W1_ff_rel, A2, c1):
  grid = NF_PAD // BLK

  def body(p1_ref, ag_ref, w_ref, a2_ref, c_ref, of1_ref, op2_ref):
    i = pl.program_id(0)

    @pl.when(i < NBLK)
    def _():
      ag = ag_ref[...].astype(jnp.float32)
      f1 = p1_ref[...] + jnp.dot(ag, w_ref[...],
                                 preferred_element_type=jnp.float32) + c_ref[...]
      f1 = jnp.maximum(f1, 0.0)
      of1_ref[...] = f1.astype(jnp.bfloat16)
      op2_ref[...] = jnp.dot(f1, a2_ref[...], preferred_element_type=jnp.float32)

    @pl.when(i >= NBLK)
    def _():
      of1_ref[...] = jnp.zeros((BLK, HID), jnp.bfloat16)
      op2_ref[...] = jnp.zeros((BLK, HID), jnp.float32)

  clamp = lambda i: (jnp.minimum(i, NBLK - 1), 0)
  return pl.pallas_call(
      body,
      grid=(grid,),
      in_specs=[
          pl.BlockSpec((BLK, HID), lambda i: (i, 0)),
          pl.BlockSpec((BLK, HID), clamp),
          pl.BlockSpec((HID, HID), lambda i: (0, 0)),
          pl.BlockSpec((HID, HID), lambda i: (0, 0)),
          pl.BlockSpec((1, HID), lambda i: (0, 0)),
      ],
      out_specs=[
          pl.BlockSpec((BLK, HID), lambda i: (i, 0)),
          pl.BlockSpec((BLK, HID), lambda i: (i, 0)),
      ],
      out_shape=[
          jax.ShapeDtypeStruct((NF_PAD, HID), jnp.bfloat16),
          jax.ShapeDtypeStruct((NF_PAD, HID), jnp.float32),
      ],
  )(P1, agg_ff1, W1_ff_rel, A2, c1)


def _host_tc(agg_fh, W1_fh_rel, b1_fh):
  HB = BLK
  grid = NH_PAD // HB

  def body(ag_ref, w_ref, b_ref, out_ref):
    i = pl.program_id(0)
    ag = ag_ref[...].astype(jnp.float32)
    h = jnp.dot(ag, w_ref[...],
                preferred_element_type=jnp.float32) + b_ref[...]
    h = jnp.maximum(h, 0.0)
    row = i * HB + lax.broadcasted_iota(jnp.int32, (HB, 1), 0)
    out_ref[...] = jnp.where(row < N_HOST, h, 0.0).astype(jnp.bfloat16)

  return pl.pallas_call(
      body,
      grid=(grid,),
      in_specs=[
          pl.BlockSpec((HB, HID), lambda i: (i, 0)),
          pl.BlockSpec((HID, HID), lambda i: (0, 0)),
          pl.BlockSpec((1, HID), lambda i: (0, 0)),
      ],
      out_specs=pl.BlockSpec((HB, HID), lambda i: (i, 0)),
      out_shape=jax.ShapeDtypeStruct((NH_PAD, HID), jnp.bfloat16),
  )(agg_fh, W1_fh_rel, b1_fh)


def _layer2_pool_tc(P2, agg_ff2, agg_hf2, Wst, c2, batch3):
  def body(p2_ref, a1_ref, a2_ref, w_ref, c_ref, b_ref, out_ref):
    i = pl.program_id(0)

    @pl.when(i == 0)
    def _():
      out_ref[...] = jnp.full((B, HID), -jnp.inf, jnp.float32)

    cat = jnp.concatenate([a1_ref[...].astype(jnp.float32),
                           a2_ref[...].astype(jnp.float32)], axis=1)
    f2 = p2_ref[...] + jnp.dot(cat, w_ref[...],
                               preferred_element_type=jnp.float32) + c_ref[...]
    bid = b_ref[0]
    lo = bid[0, 0]
    hi = bid[BLK - 1, 0]
    seg_iota = lax.broadcasted_iota(jnp.int32, (B, 1), 0)

    def seg_body(s, carry):
      m = jnp.max(jnp.where(bid == s, f2, -jnp.inf), axis=0, keepdims=True)
      upd = jnp.where(seg_iota == s, m, -jnp.inf)
      out_ref[...] = jnp.maximum(out_ref[...], upd)
      return carry

    lax.fori_loop(lo, hi + 1, seg_body, 0)

  return pl.pallas_call(
      body,
      grid=(NBLK,),
      in_specs=[
          pl.BlockSpec((BLK, HID), lambda i: (i, 0)),
          pl.BlockSpec((BLK, HID), lambda i: (i, 0)),
          pl.BlockSpec((BLK, HID), lambda i: (i, 0)),
          pl.BlockSpec((2 * HID, HID), lambda i: (0, 0)),
          pl.BlockSpec((1, HID), lambda i: (0, 0)),
          pl.BlockSpec((1, BLK, 1), lambda i: (i, 0, 0)),
      ],
      out_specs=pl.BlockSpec((B, HID), lambda i: (0, 0)),
      out_shape=jax.ShapeDtypeStruct((B, HID), jnp.float32),
  )(P2, agg_ff2, agg_hf2, Wst, c2, batch3)


def _mlp_tc(pooled, Wc1, bc1, Wc2, bc2, Wc3p, bc3p):
  def body(p_ref, w1_ref, b1_ref, w2_ref, b2_ref, w3_ref, b3_ref, out_ref):
    o = jnp.dot(p_ref[...], w1_ref[...], preferred_element_type=jnp.float32) + b1_ref[...]
    o = jnp.maximum(o, 0.0)
    o = jnp.dot(o, w2_ref[...], preferred_element_type=jnp.float32) + b2_ref[...]
    o = jnp.maximum(o, 0.0)
    out_ref[...] = jnp.dot(o, w3_ref[...], preferred_element_type=jnp.float32) + b3_ref[...]

  return pl.pallas_call(
      body,
      out_shape=jax.ShapeDtypeStruct((B, HID), jnp.float32),
  )(pooled, Wc1, bc1, Wc2, bc2, Wc3p, bc3p)


def _pad_edges(src, dst, pad_src):
  npad = E_PAD - E
  src_p = jnp.concatenate(
      [src.astype(jnp.int32), jnp.full((npad,), pad_src, jnp.int32)])
  dst_p = jnp.concatenate(
      [dst.astype(jnp.int32), jnp.zeros((npad,), jnp.int32)])
  return src_p, dst_p


def kernel(x_flow, dst_ports, tcp_flags, tcp_flags_rev, ehf_src, ehf_dst,
           efh_src, efh_dst, eff_src, eff_dst, batch, emb_port, emb_tcp,
           emb_tcp_rev, W1_hf_rel, W1_hf_root, W1_fh_rel, W1_fh_root,
           W1_ff_rel, W1_ff_root, W2_hf_rel, W2_hf_root, W2_fh_rel,
           W2_fh_root, W2_ff_rel, W2_ff_root, b1_hf, b1_fh, b1_ff, b2_hf,
           b2_fh, b2_ff, Wc1, bc1, Wc2, bc2, Wc3, bc3):
  f32 = jnp.float32

  A1 = (W1_hf_root + W1_ff_root).astype(f32)
  c1 = (b1_hf + b1_ff).reshape(1, HID).astype(f32)
  A2 = (W2_hf_root + W2_ff_root).astype(f32)
  c2 = (b2_hf + b2_ff).reshape(1, HID).astype(f32)
  Wst = jnp.concatenate([W2_ff_rel, W2_hf_rel], axis=0).astype(f32)
  Wc3p = jnp.zeros((HID, HID), f32).at[:, :NC].set(Wc3)
  bc3p = jnp.zeros((1, HID), f32).at[0, :NC].set(bc3)

  ports_p = jnp.concatenate([
      dst_ports.astype(jnp.int32),
      jnp.zeros((PG_TOTAL - N_FLOW,), jnp.int32)])
  tcp3 = tcp_flags.astype(jnp.int32).reshape(NBLK, BLK, 1)
  tcp_rev3 = tcp_flags_rev.astype(jnp.int32).reshape(NBLK, BLK, 1)
  batch3 = batch.astype(jnp.int32).reshape(NBLK, BLK, 1)

  eff_s3, eff_d3 = _pad_edges(eff_src, eff_dst, N_FLOW)
  efh_s3, efh_d3 = _pad_edges(efh_src, efh_dst, N_FLOW)
  ehf_s3, ehf_d3 = _pad_edges(ehf_src, ehf_dst, N_HOST)

  port4 = _make_port_gather()(ports_p, emb_port)
  port_rows = port4.reshape(PG_TOTAL, 16)

  xf_bf, P1 = _embed_tc(x_flow, port_rows, tcp3, tcp_rev3,
                        emb_tcp, emb_tcp_rev, A1)
  xf4 = xf_bf.reshape(NF_PAD * 4, 32)
  xf2 = xf_bf.reshape(NF_PAD * 2, 64)

  zh32 = jnp.zeros((N_FLOW // 16, 32), jnp.bfloat16)
  zh64 = jnp.zeros((NH_PAD // 16, 64), jnp.bfloat16)

  scat_flow = _make_scatter(N_FLOW, 32)
  scat_host = _make_scatter(NH_PAD, 64)
  agg_ff1 = scat_flow(eff_s3 * 4, eff_d3, xf4, zh32).reshape(N_FLOW, HID)
  agg_fh = scat_host(efh_s3 * 2, efh_d3, xf2, zh64).reshape(NH_PAD, HID)

  f1_bf, P2 = _layer1_tc(P1, agg_ff1, W1_ff_rel, A2, c1)
  h1_bf = _host_tc(agg_fh, W1_fh_rel, b1_fh.reshape(1, HID))

  f14 = f1_bf.reshape(NF_PAD * 4, 32)
  h14 = h1_bf.reshape(NH_PAD * 4, 32)
  agg_ff2 = scat_flow(eff_s3 * 4, eff_d3, f14, zh32).reshape(N_FLOW, HID)
  agg_hf2 = scat_flow(ehf_s3 * 4, ehf_d3, h14, zh32).reshape(N_FLOW, HID)

  pooled = _layer2_pool_tc(P2, agg_ff2, agg_hf2, Wst, c2, batch3)
  out = _mlp_tc(pooled, Wc1, bc1.reshape(1, HID), Wc2, bc2.reshape(1, HID),
                Wc3p, bc3p)
  return out[:, :NC]


# speedup vs baseline: 4.1950x; 1.5450x over previous
import functools

import jax
import jax.numpy as jnp
from jax import lax
from jax.experimental import pallas as pl
from jax.experimental.pallas import tpu as pltpu
from jax.experimental.pallas import tpu_sc as plsc

N_FLOW = 50000
N_HOST = 5000
E = 400000
FDIM = 108
HID = 128
NC = 10
B = 64

BLK = 2000
NBLK = N_FLOW // BLK
NF_PAD = (NBLK + 1) * BLK
NH_PAD = 6000

EB = 1280
NBATCH = 20
EPT = NBATCH * EB
E_PAD = 16 * EPT

PG_PER = 1664
PG_TOTAL = 32 * PG_PER

_mesh = lambda: plsc.VectorSubcoreMesh(core_axis_name="c", subcore_axis_name="s")
_SC_PARAMS = pltpu.CompilerParams(use_tc_tiling_on_sc=False)


def _make_port_gather():
  @functools.partial(
      pl.kernel,
      mesh=_mesh(),
      out_type=jax.ShapeDtypeStruct((32, PG_PER, 16), jnp.float32),
      compiler_params=_SC_PARAMS,
      scratch_types=[
          pltpu.VMEM((PG_PER,), jnp.int32),
          pltpu.VMEM((PG_PER, 16), jnp.float32),
          pltpu.SemaphoreType.DMA,
      ],
  )
  def gather_k(ports1, embp, out3, ibuf, gbuf, sem):
    cid = lax.axis_index("c")
    sid = lax.axis_index("s")
    wid = cid * 16 + sid
    pltpu.sync_copy(ports1.at[pl.ds(wid * PG_PER, PG_PER)], ibuf)
    pltpu.async_copy(embp.at[ibuf], gbuf, sem).wait()
    pltpu.sync_copy(gbuf, out3.at[wid])

  return gather_k


def _make_scatter(n_dst, cw):
  rpt = n_dst // 16
  nch = 128 // cw
  kpc = nch // 2

  @functools.partial(
      pl.kernel,
      mesh=_mesh(),
      out_type=jax.ShapeDtypeStruct((n_dst, nch, cw), jnp.bfloat16),
      compiler_params=_SC_PARAMS,
      scratch_types=[
          pltpu.VMEM((EB,), jnp.int32),
          pltpu.VMEM((EB,), jnp.int32),
          pltpu.VMEM((EB,), jnp.int32),
          pltpu.VMEM((EB,), jnp.int32),
          pltpu.VMEM((EB, cw), jnp.bfloat16),
          pltpu.VMEM((EB, cw), jnp.bfloat16),
          pltpu.VMEM_SHARED((n_dst, cw), jnp.bfloat16),
          pltpu.SemaphoreType.DMA,
          pltpu.SemaphoreType.DMA,
          pltpu.SemaphoreType.DMA,
          pltpu.SemaphoreType.DMA,
      ],
  )
  def scatter_k(srcv, dst1, xv, zhbm, out3, gi0, gi1, db0, db1, gb0, gb1,
                acc, sg0, sg1, ss0, ss1):
    cid = lax.axis_index("c")
    sid = lax.axis_index("s")
    edge_base = sid * EPT

    gidxs, dbufs = (gi0, gi1), (db0, db1)
    gbufs, gsems, ssems = (gb0, gb1), (sg0, sg1), (ss0, ss1)

    for k in range(kpc):
      c = cid * kpc + k

      pltpu.sync_copy(zhbm.at[pl.ds(0, rpt)], acc.at[pl.ds(sid * rpt, rpt)])
      plsc.subcore_barrier()

      def load_and_start_gather(b, slot):
        base = edge_base + b * EB
        pltpu.sync_copy(srcv.at[pl.ds(base, EB)], gidxs[slot])
        pltpu.sync_copy(dst1.at[pl.ds(base, EB)], dbufs[slot])
        for j in range(EB // 16):
          v = gidxs[slot][pl.ds(j * 16, 16)]
          gidxs[slot][pl.ds(j * 16, 16)] = v + c
        pltpu.async_copy(xv.at[gidxs[slot]], gbufs[slot], gsems[slot])

      def wait_gather(slot):
        pltpu.make_async_copy(xv.at[gidxs[slot]], gbufs[slot],
                              gsems[slot]).wait()

      def scatter_desc(slot):
        return pltpu.make_async_copy(gbufs[slot], acc.at[dbufs[slot]],
                                     ssems[slot])

      load_and_start_gather(0, 0)

      def pair_body(i, carry):
        for slot in range(2):
          other = 1 - slot
          b = 2 * i + slot
          wait_gather(slot)
          scatter_desc(slot).start(add=True)

          @pl.when(b + 1 < NBATCH)
          def _():
            @pl.when(b >= 1)
            def _():
              scatter_desc(other).wait()
            load_and_start_gather(b + 1, other)
        return carry

      lax.fori_loop(0, NBATCH // 2, pair_body, 0)
      scatter_desc(0).wait()
      scatter_desc(1).wait()
      plsc.subcore_barrier()

      wb = sid * rpt
      pltpu.sync_copy(acc.at[pl.ds(wb, rpt)], out3.at[pl.ds(wb, rpt), c])
      plsc.subcore_barrier()

  return scatter_k


def _embed_tc(x_flow, port_rows, tcp3, tcp_rev3, et1, et2, A1):
  grid = NF_PAD // BLK

  def body(xf_ref, pr_ref, t1_ref, t2_ref, e1_ref, e2_ref, a1_ref,
           oxf_ref, op1_ref):
    i = pl.program_id(0)

    @pl.when(i < NBLK)
    def _():
      t1 = t1_ref[0]
      t2 = t2_ref[0]
      io = lax.broadcasted_iota(jnp.int32, (1, 256), 1)
      oh1 = (t1 == io).astype(jnp.float32)
      oh2 = (t2 == io).astype(jnp.float32)
      e1 = jnp.dot(oh1, e1_ref[...], preferred_element_type=jnp.float32)
      e2 = jnp.dot(oh2, e2_ref[...], preferred_element_type=jnp.float32)
      xf = jnp.concatenate([xf_ref[...], pr_ref[...], e1, e2], axis=1)
      oxf_ref[...] = xf.astype(jnp.bfloat16)
      op1_ref[...] = jnp.dot(xf, a1_ref[...], preferred_element_type=jnp.float32)

    @pl.when(i >= NBLK)
    def _():
      oxf_ref[...] = jnp.zeros((BLK, HID), jnp.bfloat16)
      op1_ref[...] = jnp.zeros((BLK, HID), jnp.float32)

  clamp = lambda i: (jnp.minimum(i, NBLK - 1), 0)
  clamp3 = lambda i: (jnp.minimum(i, NBLK - 1), 0, 0)
  out = pl.pallas_call(
      body,
      grid=(grid,),
      in_specs=[
          pl.BlockSpec((BLK, FDIM), clamp),
          pl.BlockSpec((BLK, 16), clamp),
          pl.BlockSpec((1, BLK, 1), clamp3),
          pl.BlockSpec((1, BLK, 1), clamp3),
          pl.BlockSpec((256, 2), lambda i: (0, 0)),
          pl.BlockSpec((256, 2), lambda i: (0, 0)),
          pl.BlockSpec((HID, HID), lambda i: (0, 0)),
      ],
      out_specs=[
          pl.BlockSpec((BLK, HID), lambda i: (i, 0)),
          pl.BlockSpec((BLK, HID), lambda i: (i, 0)),
      ],
      out_shape=[
          jax.ShapeDtypeStruct((NF_PAD, HID), jnp.bfloat16),
          jax.ShapeDtypeStruct((NF_PAD, HID), jnp.float32),
      ],
  )(x_flow, port_rows, tcp3, tcp_rev3, et1, et2, A1)
  return out


def _layer1_tc(P1, agg_ff1, W1_ff_rel, A2, c1):
  grid = NF_PAD // BLK

  def body(p1_ref, ag_ref, w_ref, a2_ref, c_ref, of1_ref, op2_ref):
    i = pl.program_id(0)

    @pl.when(i < NBLK)
    def _():
      ag = ag_ref[...].astype(jnp.float32)
      f1 = p1_ref[...] + jnp.dot(ag, w_ref[...],
                                 preferred_element_type=jnp.float32) + c_ref[...]
      f1 = jnp.maximum(f1, 0.0)
      of1_ref[...] = f1.astype(jnp.bfloat16)
      op2_ref[...] = jnp.dot(f1, a2_ref[...], preferred_element_type=jnp.float32)

    @pl.when(i >= NBLK)
    def _():
      of1_ref[...] = jnp.zeros((BLK, HID), jnp.bfloat16)
      op2_ref[...] = jnp.zeros((BLK, HID), jnp.float32)

  clamp = lambda i: (jnp.minimum(i, NBLK - 1), 0)
  return pl.pallas_call(
      body,
      grid=(grid,),
      in_specs=[
          pl.BlockSpec((BLK, HID), lambda i: (i, 0)),
          pl.BlockSpec((BLK, HID), clamp),
          pl.BlockSpec((HID, HID), lambda i: (0, 0)),
          pl.BlockSpec((HID, HID), lambda i: (0, 0)),
          pl.BlockSpec((1, HID), lambda i: (0, 0)),
      ],
      out_specs=[
          pl.BlockSpec((BLK, HID), lambda i: (i, 0)),
          pl.BlockSpec((BLK, HID), lambda i: (i, 0)),
      ],
      out_shape=[
          jax.ShapeDtypeStruct((NF_PAD, HID), jnp.bfloat16),
          jax.ShapeDtypeStruct((NF_PAD, HID), jnp.float32),
      ],
  )(P1, agg_ff1, W1_ff_rel, A2, c1)


def _host_tc(agg_fh, W1_fh_rel, b1_fh):
  HB = BLK
  grid = NH_PAD // HB

  def body(ag_ref, w_ref, b_ref, out_ref):
    i = pl.program_id(0)
    ag = ag_ref[...].astype(jnp.float32)
    h = jnp.dot(ag, w_ref[...],
                preferred_element_type=jnp.float32) + b_ref[...]
    h = jnp.maximum(h, 0.0)
    row = i * HB + lax.broadcasted_iota(jnp.int32, (HB, 1), 0)
    out_ref[...] = jnp.where(row < N_HOST, h, 0.0).astype(jnp.bfloat16)

  return pl.pallas_call(
      body,
      grid=(grid,),
      in_specs=[
          pl.BlockSpec((HB, HID), lambda i: (i, 0)),
          pl.BlockSpec((HID, HID), lambda i: (0, 0)),
          pl.BlockSpec((1, HID), lambda i: (0, 0)),
      ],
      out_specs=pl.BlockSpec((HB, HID), lambda i: (i, 0)),
      out_shape=jax.ShapeDtypeStruct((NH_PAD, HID), jnp.bfloat16),
  )(agg_fh, W1_fh_rel, b1_fh)


def _layer2_pool_tc(P2, agg_ff2, agg_hf2, Wst, c2, batch3):
  def body(p2_ref, a1_ref, a2_ref, w_ref, c_ref, b_ref, out_ref):
    i = pl.program_id(0)

    @pl.when(i == 0)
    def _():
      out_ref[...] = jnp.full((B, HID), -jnp.inf, jnp.float32)

    cat = jnp.concatenate([a1_ref[...].astype(jnp.float32),
                           a2_ref[...].astype(jnp.float32)], axis=1)
    f2 = p2_ref[...] + jnp.dot(cat, w_ref[...],
                               preferred_element_type=jnp.float32) + c_ref[...]
    bid = b_ref[0]
    lo = bid[0, 0]
    hi = bid[BLK - 1, 0]
    seg_iota = lax.broadcasted_iota(jnp.int32, (B, 1), 0)

    def seg_body(s, carry):
      m = jnp.max(jnp.where(bid == s, f2, -jnp.inf), axis=0, keepdims=True)
      upd = jnp.where(seg_iota == s, m, -jnp.inf)
      out_ref[...] = jnp.maximum(out_ref[...], upd)
      return carry

    lax.fori_loop(lo, hi + 1, seg_body, 0)

  return pl.pallas_call(
      body,
      grid=(NBLK,),
      in_specs=[
          pl.BlockSpec((BLK, HID), lambda i: (i, 0)),
          pl.BlockSpec((BLK, HID), lambda i: (i, 0)),
          pl.BlockSpec((BLK, HID), lambda i: (i, 0)),
          pl.BlockSpec((2 * HID, HID), lambda i: (0, 0)),
          pl.BlockSpec((1, HID), lambda i: (0, 0)),
          pl.BlockSpec((1, BLK, 1), lambda i: (i, 0, 0)),
      ],
      out_specs=pl.BlockSpec((B, HID), lambda i: (0, 0)),
      out_shape=jax.ShapeDtypeStruct((B, HID), jnp.float32),
  )(P2, agg_ff2, agg_hf2, Wst, c2, batch3)


def _mlp_tc(pooled, Wc1, bc1, Wc2, bc2, Wc3p, bc3p):
  def body(p_ref, w1_ref, b1_ref, w2_ref, b2_ref, w3_ref, b3_ref, out_ref):
    o = jnp.dot(p_ref[...], w1_ref[...], preferred_element_type=jnp.float32) + b1_ref[...]
    o = jnp.maximum(o, 0.0)
    o = jnp.dot(o, w2_ref[...], preferred_element_type=jnp.float32) + b2_ref[...]
    o = jnp.maximum(o, 0.0)
    out_ref[...] = jnp.dot(o, w3_ref[...], preferred_element_type=jnp.float32) + b3_ref[...]

  return pl.pallas_call(
      body,
      out_shape=jax.ShapeDtypeStruct((B, HID), jnp.float32),
  )(pooled, Wc1, bc1, Wc2, bc2, Wc3p, bc3p)


def _pad_edges(src, dst, pad_src):
  npad = E_PAD - E
  src_p = jnp.concatenate(
      [src.astype(jnp.int32), jnp.full((npad,), pad_src, jnp.int32)])
  dst_p = jnp.concatenate(
      [dst.astype(jnp.int32), jnp.zeros((npad,), jnp.int32)])
  return src_p, dst_p


def kernel(x_flow, dst_ports, tcp_flags, tcp_flags_rev, ehf_src, ehf_dst,
           efh_src, efh_dst, eff_src, eff_dst, batch, emb_port, emb_tcp,
           emb_tcp_rev, W1_hf_rel, W1_hf_root, W1_fh_rel, W1_fh_root,
           W1_ff_rel, W1_ff_root, W2_hf_rel, W2_hf_root, W2_fh_rel,
           W2_fh_root, W2_ff_rel, W2_ff_root, b1_hf, b1_fh, b1_ff, b2_hf,
           b2_fh, b2_ff, Wc1, bc1, Wc2, bc2, Wc3, bc3):
  f32 = jnp.float32

  A1 = (W1_hf_root + W1_ff_root).astype(f32)
  c1 = (b1_hf + b1_ff).reshape(1, HID).astype(f32)
  A2 = (W2_hf_root + W2_ff_root).astype(f32)
  c2 = (b2_hf + b2_ff).reshape(1, HID).astype(f32)
  Wst = jnp.concatenate([W2_ff_rel, W2_hf_rel], axis=0).astype(f32)
  Wc3p = jnp.zeros((HID, HID), f32).at[:, :NC].set(Wc3)
  bc3p = jnp.zeros((1, HID), f32).at[0, :NC].set(bc3)

  ports_p = jnp.concatenate([
      dst_ports.astype(jnp.int32),
      jnp.zeros((PG_TOTAL - N_FLOW,), jnp.int32)])
  tcp3 = tcp_flags.astype(jnp.int32).reshape(NBLK, BLK, 1)
  tcp_rev3 = tcp_flags_rev.astype(jnp.int32).reshape(NBLK, BLK, 1)
  batch3 = batch.astype(jnp.int32).reshape(NBLK, BLK, 1)

  eff_s3, eff_d3 = _pad_edges(eff_src, eff_dst, N_FLOW)
  efh_s3, efh_d3 = _pad_edges(efh_src, efh_dst, N_FLOW)
  ehf_s3, ehf_d3 = _pad_edges(ehf_src, ehf_dst, N_HOST)

  port4 = _make_port_gather()(ports_p, emb_port)
  port_rows = port4.reshape(PG_TOTAL, 16)

  xf_bf, P1 = _embed_tc(x_flow, port_rows, tcp3, tcp_rev3,
                        emb_tcp, emb_tcp_rev, A1)
  xf4 = xf_bf.reshape(NF_PAD * 4, 32)
  xf2 = xf_bf.reshape(NF_PAD * 2, 64)

  zh32 = jnp.zeros((N_FLOW // 16, 32), jnp.bfloat16)
  zh64 = jnp.zeros((NH_PAD // 16, 64), jnp.bfloat16)

  scat_flow = _make_scatter(N_FLOW, 32)
  scat_host = _make_scatter(NH_PAD, 64)
  agg_ff1 = scat_flow(eff_s3 * 4, eff_d3, xf4, zh32).reshape(N_FLOW, HID)
  agg_fh = scat_host(efh_s3 * 2, efh_d3, xf2, zh64).reshape(NH_PAD, HID)

  f1_bf, P2 = _layer1_tc(P1, agg_ff1, W1_ff_rel, A2, c1)
  h1_bf = _host_tc(agg_fh, W1_fh_rel, b1_fh.reshape(1, HID))

  f14 = f1_bf.reshape(NF_PAD * 4, 32)
  h14 = h1_bf.reshape(NH_PAD * 4, 32)
  agg_ff2 = scat_flow(eff_s3 * 4, eff_d3, f14, zh32).reshape(N_FLOW, HID)
  agg_hf2 = scat_flow(ehf_s3 * 4, ehf_d3, h14, zh32).reshape(N_FLOW, HID)

  pooled = _layer2_pool_tc(P2, agg_ff2, agg_hf2, Wst, c2, batch3)
  out = _mlp_tc(pooled, Wc1, bc1.reshape(1, HID), Wc2, bc2.reshape(1, HID),
                Wc3p, bc3p)
  return out[:, :NC]


# speedup vs baseline: 4.2615x; 1.0159x over previous
import functools

import jax
import jax.numpy as jnp
from jax import lax
from jax.experimental import pallas as pl
from jax.experimental.pallas import tpu as pltpu
from jax.experimental.pallas import tpu_sc as plsc

N_FLOW = 50000
N_HOST = 5000
E = 400000
FDIM = 108
HID = 128
NC = 10
B = 64

BLK = 2000
NBLK = N_FLOW // BLK
NF_PAD = (NBLK + 1) * BLK
NH_PAD = 6000

EPT = 25600
E_PAD = 16 * EPT

PG_PER = 1664
PG_TOTAL = 32 * PG_PER

_mesh = lambda: plsc.VectorSubcoreMesh(core_axis_name="c", subcore_axis_name="s")
_SC_PARAMS = pltpu.CompilerParams(use_tc_tiling_on_sc=False)


def _make_port_gather():
  @functools.partial(
      pl.kernel,
      mesh=_mesh(),
      out_type=jax.ShapeDtypeStruct((32, PG_PER, 16), jnp.float32),
      compiler_params=_SC_PARAMS,
      scratch_types=[
          pltpu.VMEM((PG_PER,), jnp.int32),
          pltpu.VMEM((PG_PER, 16), jnp.float32),
          pltpu.SemaphoreType.DMA,
      ],
  )
  def gather_k(ports1, embp, out3, ibuf, gbuf, sem):
    cid = lax.axis_index("c")
    sid = lax.axis_index("s")
    wid = cid * 16 + sid
    pltpu.sync_copy(ports1.at[pl.ds(wid * PG_PER, PG_PER)], ibuf)
    pltpu.async_copy(embp.at[ibuf], gbuf, sem).wait()
    pltpu.sync_copy(gbuf, out3.at[wid])

  return gather_k


def _make_scatter(n_dst, cw):
  rpt = n_dst // 16
  nch = 128 // cw
  kpc = nch // 2
  EB = 1600
  NBATCH = EPT // EB

  @functools.partial(
      pl.kernel,
      mesh=_mesh(),
      out_type=jax.ShapeDtypeStruct((n_dst, nch, cw), jnp.bfloat16),
      compiler_params=_SC_PARAMS,
      scratch_types=[
          pltpu.VMEM((EB,), jnp.int32),
          pltpu.VMEM((EB,), jnp.int32),
          pltpu.VMEM((EB,), jnp.int32),
          pltpu.VMEM((EB,), jnp.int32),
          pltpu.VMEM((EB, cw), jnp.bfloat16),
          pltpu.VMEM((EB, cw), jnp.bfloat16),
          pltpu.VMEM_SHARED((n_dst, cw), jnp.bfloat16),
          pltpu.SemaphoreType.DMA,
          pltpu.SemaphoreType.DMA,
          pltpu.SemaphoreType.DMA,
          pltpu.SemaphoreType.DMA,
      ],
  )
  def scatter_k(srcv, dst1, xv, zhbm, out3, gi0, gi1, db0, db1, gb0, gb1,
                acc, sg0, sg1, ss0, ss1):
    cid = lax.axis_index("c")
    sid = lax.axis_index("s")
    edge_base = sid * EPT

    gidxs, dbufs = (gi0, gi1), (db0, db1)
    gbufs, gsems, ssems = (gb0, gb1), (sg0, sg1), (ss0, ss1)

    for k in range(kpc):
      c = cid * kpc + k

      pltpu.sync_copy(zhbm.at[pl.ds(0, rpt)], acc.at[pl.ds(sid * rpt, rpt)])
      plsc.subcore_barrier()

      def load_and_start_gather(b, slot):
        base = edge_base + b * EB
        pltpu.sync_copy(srcv.at[pl.ds(base, EB)], gidxs[slot])
        pltpu.sync_copy(dst1.at[pl.ds(base, EB)], dbufs[slot])
        for j in range(EB // 16):
          v = gidxs[slot][pl.ds(j * 16, 16)]
          gidxs[slot][pl.ds(j * 16, 16)] = v + c
        pltpu.async_copy(xv.at[gidxs[slot]], gbufs[slot], gsems[slot])

      def wait_gather(slot):
        pltpu.make_async_copy(xv.at[gidxs[slot]], gbufs[slot],
                              gsems[slot]).wait()

      def scatter_desc(slot):
        return pltpu.make_async_copy(gbufs[slot], acc.at[dbufs[slot]],
                                     ssems[slot])

      load_and_start_gather(0, 0)

      def pair_body(i, carry):
        for slot in range(2):
          other = 1 - slot
          b = 2 * i + slot
          wait_gather(slot)
          scatter_desc(slot).start(add=True)

          @pl.when(b + 1 < NBATCH)
          def _():
            @pl.when(b >= 1)
            def _():
              scatter_desc(other).wait()
            load_and_start_gather(b + 1, other)
        return carry

      lax.fori_loop(0, NBATCH // 2, pair_body, 0)
      scatter_desc(0).wait()
      scatter_desc(1).wait()
      plsc.subcore_barrier()

      wb = sid * rpt
      pltpu.sync_copy(acc.at[pl.ds(wb, rpt)], out3.at[pl.ds(wb, rpt), c])
      plsc.subcore_barrier()

  return scatter_k


def _embed_tc(x_flow, port_rows, tcp3, tcp_rev3, et1, et2, A1):
  grid = NF_PAD // BLK

  def body(xf_ref, pr_ref, t1_ref, t2_ref, e1_ref, e2_ref, a1_ref,
           oxf_ref, op1_ref):
    i = pl.program_id(0)

    @pl.when(i < NBLK)
    def _():
      t1 = t1_ref[0]
      t2 = t2_ref[0]
      io = lax.broadcasted_iota(jnp.int32, (1, 256), 1)
      oh1 = (t1 == io).astype(jnp.float32)
      oh2 = (t2 == io).astype(jnp.float32)
      e1 = jnp.dot(oh1, e1_ref[...], preferred_element_type=jnp.float32)
      e2 = jnp.dot(oh2, e2_ref[...], preferred_element_type=jnp.float32)
      xf = jnp.concatenate([xf_ref[...], pr_ref[...], e1, e2], axis=1)
      oxf_ref[...] = xf.astype(jnp.bfloat16)
      op1_ref[...] = jnp.dot(xf, a1_ref[...], preferred_element_type=jnp.float32)

    @pl.when(i >= NBLK)
    def _():
      oxf_ref[...] = jnp.zeros((BLK, HID), jnp.bfloat16)
      op1_ref[...] = jnp.zeros((BLK, HID), jnp.float32)

  clamp = lambda i: (jnp.minimum(i, NBLK - 1), 0)
  clamp3 = lambda i: (jnp.minimum(i, NBLK - 1), 0, 0)
  out = pl.pallas_call(
      body,
      grid=(grid,),
      in_specs=[
          pl.BlockSpec((BLK, FDIM), clamp),
          pl.BlockSpec((BLK, 16), clamp),
          pl.BlockSpec((1, BLK, 1), clamp3),
          pl.BlockSpec((1, BLK, 1), clamp3),
          pl.BlockSpec((256, 2), lambda i: (0, 0)),
          pl.BlockSpec((256, 2), lambda i: (0, 0)),
          pl.BlockSpec((HID, HID), lambda i: (0, 0)),
      ],
      out_specs=[
          pl.BlockSpec((BLK, HID), lambda i: (i, 0)),
          pl.BlockSpec((BLK, HID), lambda i: (i, 0)),
      ],
      out_shape=[
          jax.ShapeDtypeStruct((NF_PAD, HID), jnp.bfloat16),
          jax.ShapeDtypeStruct((NF_PAD, HID), jnp.float32),
      ],
  )(x_flow, port_rows, tcp3, tcp_rev3, et1, et2, A1)
  return out


def _layer1_tc(P1, agg_ff1, W1_ff_rel, A2, c1):
  grid = NF_PAD // BLK

  def body(p1_ref, ag_ref, w_ref, a2_ref, c_ref, of1_ref, op2_ref):
    i = pl.program_id(0)

    @pl.when(i < NBLK)
    def _():
      ag = ag_ref[...].astype(jnp.float32)
      f1 = p1_ref[...] + jnp.dot(ag, w_ref[...],
                                 preferred_element_type=jnp.float32) + c_ref[...]
      f1 = jnp.maximum(f1, 0.0)
      of1_ref[...] = f1.astype(jnp.bfloat16)
      op2_ref[...] = jnp.dot(f1, a2_ref[...], preferred_element_type=jnp.float32)

    @pl.when(i >= NBLK)
    def _():
      of1_ref[...] = jnp.zeros((BLK, HID), jnp.bfloat16)
      op2_ref[...] = jnp.zeros((BLK, HID), jnp.float32)

  clamp = lambda i: (jnp.minimum(i, NBLK - 1), 0)
  return pl.pallas_call(
      body,
      grid=(grid,),
      in_specs=[
          pl.BlockSpec((BLK, HID), lambda i: (i, 0)),
          pl.BlockSpec((BLK, HID), clamp),
          pl.BlockSpec((HID, HID), lambda i: (0, 0)),
          pl.BlockSpec((HID, HID), lambda i: (0, 0)),
          pl.BlockSpec((1, HID), lambda i: (0, 0)),
      ],
      out_specs=[
          pl.BlockSpec((BLK, HID), lambda i: (i, 0)),
          pl.BlockSpec((BLK, HID), lambda i: (i, 0)),
      ],
      out_shape=[
          jax.ShapeDtypeStruct((NF_PAD, HID), jnp.bfloat16),
          jax.ShapeDtypeStruct((NF_PAD, HID), jnp.float32),
      ],
  )(P1, agg_ff1, W1_ff_rel, A2, c1)


def _host_tc(agg_fh, W1_fh_rel, b1_fh):
  HB = BLK
  grid = NH_PAD // HB

  def body(ag_ref, w_ref, b_ref, out_ref):
    i = pl.program_id(0)
    ag = ag_ref[...].astype(jnp.float32)
    h = jnp.dot(ag, w_ref[...],
                preferred_element_type=jnp.float32) + b_ref[...]
    h = jnp.maximum(h, 0.0)
    row = i * HB + lax.broadcasted_iota(jnp.int32, (HB, 1), 0)
    out_ref[...] = jnp.where(row < N_HOST, h, 0.0).astype(jnp.bfloat16)

  return pl.pallas_call(
      body,
      grid=(grid,),
      in_specs=[
          pl.BlockSpec((HB, HID), lambda i: (i, 0)),
          pl.BlockSpec((HID, HID), lambda i: (0, 0)),
          pl.BlockSpec((1, HID), lambda i: (0, 0)),
      ],
      out_specs=pl.BlockSpec((HB, HID), lambda i: (i, 0)),
      out_shape=jax.ShapeDtypeStruct((NH_PAD, HID), jnp.bfloat16),
  )(agg_fh, W1_fh_rel, b1_fh)


def _layer2_pool_tc(P2, agg_ff2, agg_hf2, Wst, c2, batch3):
  def body(p2_ref, a1_ref, a2_ref, w_ref, c_ref, b_ref, out_ref):
    i = pl.program_id(0)

    @pl.when(i == 0)
    def _():
      out_ref[...] = jnp.full((B, HID), -jnp.inf, jnp.float32)

    cat = jnp.concatenate([a1_ref[...].astype(jnp.float32),
                           a2_ref[...].astype(jnp.float32)], axis=1)
    f2 = p2_ref[...] + jnp.dot(cat, w_ref[...],
                               preferred_element_type=jnp.float32) + c_ref[...]
    bid = b_ref[0]
    lo = bid[0, 0]
    hi = bid[BLK - 1, 0]
    seg_iota = lax.broadcasted_iota(jnp.int32, (B, 1), 0)

    def seg_body(s, carry):
      m = jnp.max(jnp.where(bid == s, f2, -jnp.inf), axis=0, keepdims=True)
      upd = jnp.where(seg_iota == s, m, -jnp.inf)
      out_ref[...] = jnp.maximum(out_ref[...], upd)
      return carry

    lax.fori_loop(lo, hi + 1, seg_body, 0)

  return pl.pallas_call(
      body,
      grid=(NBLK,),
      in_specs=[
          pl.BlockSpec((BLK, HID), lambda i: (i, 0)),
          pl.BlockSpec((BLK, HID), lambda i: (i, 0)),
          pl.BlockSpec((BLK, HID), lambda i: (i, 0)),
          pl.BlockSpec((2 * HID, HID), lambda i: (0, 0)),
          pl.BlockSpec((1, HID), lambda i: (0, 0)),
          pl.BlockSpec((1, BLK, 1), lambda i: (i, 0, 0)),
      ],
      out_specs=pl.BlockSpec((B, HID), lambda i: (0, 0)),
      out_shape=jax.ShapeDtypeStruct((B, HID), jnp.float32),
  )(P2, agg_ff2, agg_hf2, Wst, c2, batch3)


def _mlp_tc(pooled, Wc1, bc1, Wc2, bc2, Wc3p, bc3p):
  def body(p_ref, w1_ref, b1_ref, w2_ref, b2_ref, w3_ref, b3_ref, out_ref):
    o = jnp.dot(p_ref[...], w1_ref[...], preferred_element_type=jnp.float32) + b1_ref[...]
    o = jnp.maximum(o, 0.0)
    o = jnp.dot(o, w2_ref[...], preferred_element_type=jnp.float32) + b2_ref[...]
    o = jnp.maximum(o, 0.0)
    out_ref[...] = jnp.dot(o, w3_ref[...], preferred_element_type=jnp.float32) + b3_ref[...]

  return pl.pallas_call(
      body,
      out_shape=jax.ShapeDtypeStruct((B, HID), jnp.float32),
  )(pooled, Wc1, bc1, Wc2, bc2, Wc3p, bc3p)


def _pad_edges(src, dst, pad_src):
  npad = E_PAD - E
  src_p = jnp.concatenate(
      [src.astype(jnp.int32), jnp.full((npad,), pad_src, jnp.int32)])
  dst_p = jnp.concatenate(
      [dst.astype(jnp.int32), jnp.zeros((npad,), jnp.int32)])
  return src_p, dst_p


def kernel(x_flow, dst_ports, tcp_flags, tcp_flags_rev, ehf_src, ehf_dst,
           efh_src, efh_dst, eff_src, eff_dst, batch, emb_port, emb_tcp,
           emb_tcp_rev, W1_hf_rel, W1_hf_root, W1_fh_rel, W1_fh_root,
           W1_ff_rel, W1_ff_root, W2_hf_rel, W2_hf_root, W2_fh_rel,
           W2_fh_root, W2_ff_rel, W2_ff_root, b1_hf, b1_fh, b1_ff, b2_hf,
           b2_fh, b2_ff, Wc1, bc1, Wc2, bc2, Wc3, bc3):
  f32 = jnp.float32

  A1 = (W1_hf_root + W1_ff_root).astype(f32)
  c1 = (b1_hf + b1_ff).reshape(1, HID).astype(f32)
  A2 = (W2_hf_root + W2_ff_root).astype(f32)
  c2 = (b2_hf + b2_ff).reshape(1, HID).astype(f32)
  Wst = jnp.concatenate([W2_ff_rel, W2_hf_rel], axis=0).astype(f32)
  Wc3p = jnp.zeros((HID, HID), f32).at[:, :NC].set(Wc3)
  bc3p = jnp.zeros((1, HID), f32).at[0, :NC].set(bc3)

  ports_p = jnp.concatenate([
      dst_ports.astype(jnp.int32),
      jnp.zeros((PG_TOTAL - N_FLOW,), jnp.int32)])
  tcp3 = tcp_flags.astype(jnp.int32).reshape(NBLK, BLK, 1)
  tcp_rev3 = tcp_flags_rev.astype(jnp.int32).reshape(NBLK, BLK, 1)
  batch3 = batch.astype(jnp.int32).reshape(NBLK, BLK, 1)

  eff_s3, eff_d3 = _pad_edges(eff_src, eff_dst, N_FLOW)
  efh_s3, efh_d3 = _pad_edges(efh_src, efh_dst, N_FLOW)
  ehf_s3, ehf_d3 = _pad_edges(ehf_src, ehf_dst, N_HOST)

  port4 = _make_port_gather()(ports_p, emb_port)
  port_rows = port4.reshape(PG_TOTAL, 16)

  xf_bf, P1 = _embed_tc(x_flow, port_rows, tcp3, tcp_rev3,
                        emb_tcp, emb_tcp_rev, A1)
  xf4 = xf_bf.reshape(NF_PAD * 4, 32)
  xf2 = xf_bf.reshape(NF_PAD * 2, 64)

  zh32 = jnp.zeros((N_FLOW // 16, 32), jnp.bfloat16)
  zh64 = jnp.zeros((NH_PAD // 16, 64), jnp.bfloat16)

  scat_flow = _make_scatter(N_FLOW, 32)
  scat_host = _make_scatter(NH_PAD, 64)
  agg_ff1 = scat_flow(eff_s3 * 4, eff_d3, xf4, zh32).reshape(N_FLOW, HID)
  agg_fh = scat_host(efh_s3 * 2, efh_d3, xf2, zh64).reshape(NH_PAD, HID)

  f1_bf, P2 = _layer1_tc(P1, agg_ff1, W1_ff_rel, A2, c1)
  h1_bf = _host_tc(agg_fh, W1_fh_rel, b1_fh.reshape(1, HID))

  f14 = f1_bf.reshape(NF_PAD * 4, 32)
  h14 = h1_bf.reshape(NH_PAD * 4, 32)
  agg_ff2 = scat_flow(eff_s3 * 4, eff_d3, f14, zh32).reshape(N_FLOW, HID)
  agg_hf2 = scat_flow(ehf_s3 * 4, ehf_d3, h14, zh32).reshape(N_FLOW, HID)

  pooled = _layer2_pool_tc(P2, agg_ff2, agg_hf2, Wst, c2, batch3)
  out = _mlp_tc(pooled, Wc1, bc1.reshape(1, HID), Wc2, bc2.reshape(1, HID),
                Wc3p, bc3p)
  return out[:, :NC]
